# baseline (speedup 1.0000x reference)
import functools

import jax
import jax.numpy as jnp
from jax.experimental import pallas as pl
from jax.experimental.pallas import tpu as pltpu


def _win5_ax0(x):
    a = x.shape[0]
    m1 = jnp.maximum(x[0:a - 1], x[1:a])
    m2 = jnp.maximum(m1[0:a - 3], m1[2:a - 1])
    return jnp.maximum(m2[0:a - 4], x[4:a])


def _win5_ax1(x):
    b = x.shape[1]
    m1 = jnp.maximum(x[:, 0:b - 1], x[:, 1:b])
    m2 = jnp.maximum(m1[:, 0:b - 3], m1[:, 2:b - 1])
    return jnp.maximum(m2[:, 0:b - 4], x[:, 4:b])


def _pool5(x):
    return _win5_ax1(_win5_ax0(x))


def _spp_kernel(h, w, x_ref, w1_ref, w2_ref, s1_ref, b1_ref, s2_ref,
                b2_ref, o_ref):
    nb = x_ref.shape[1]
    c1 = x_ref.shape[2]
    m = h * w * nb
    xm = x_ref[...].reshape(m, c1)
    y = jnp.dot(xm, w1_ref[...], preferred_element_type=jnp.float32)
    y = y + b1_ref[...]
    y = y * jax.nn.sigmoid(y)
    c = y.shape[-1]

    y4 = y.reshape(h, w, nb, c)
    neg_rows = jnp.full((6, w, nb, c), -jnp.inf, jnp.float32)
    yp = jnp.concatenate([neg_rows, y4, neg_rows], axis=0)
    neg_cols = jnp.full((h + 12, 6, nb, c), -jnp.inf, jnp.float32)
    yp = jnp.concatenate([neg_cols, yp, neg_cols], axis=1)
    q1 = _pool5(yp)
    q2 = _pool5(q1)
    q3 = _pool5(q2)
    p5 = q1[4:4 + h, 4:4 + w].reshape(m, c)
    p9 = q2[2:2 + h, 2:2 + w].reshape(m, c)
    p13 = q3.reshape(m, c)

    acc = jnp.dot(y, w2_ref[0], preferred_element_type=jnp.float32)
    acc = acc + jnp.dot(p5, w2_ref[1], preferred_element_type=jnp.float32)
    acc = acc + jnp.dot(p9, w2_ref[2], preferred_element_type=jnp.float32)
    acc = acc + jnp.dot(p13, w2_ref[3], preferred_element_type=jnp.float32)
    z = acc + b2_ref[...]
    z = z * jax.nn.sigmoid(z)
    o_ref[...] = z.reshape(o_ref.shape).astype(o_ref.dtype)


@jax.jit
def kernel(x, w1, s1, b1, w2, s2, b2):
    n, c1, h, w = x.shape
    cp = w1.shape[1]
    c2 = w2.shape[1]
    hw = h * w
    nb = 8
    while n % nb:
        nb -= 1
    xv = jnp.transpose(x, (2, 3, 0, 1)).reshape(hw, n, c1)
    w1 = w1 * s1.reshape(1, cp)
    w2r = w2.reshape(4, cp, c2) * s2.reshape(1, 1, c2)
    out = pl.pallas_call(
        functools.partial(_spp_kernel, h, w),
        out_shape=jax.ShapeDtypeStruct((hw, n, c2), x.dtype),
        grid=(n // nb,),
        in_specs=[
            pl.BlockSpec((hw, nb, c1), lambda i: (0, i, 0)),
            pl.BlockSpec((c1, cp), lambda i: (0, 0)),
            pl.BlockSpec((4, cp, c2), lambda i: (0, 0, 0)),
            pl.BlockSpec((1, cp), lambda i: (0, 0)),
            pl.BlockSpec((1, cp), lambda i: (0, 0)),
            pl.BlockSpec((1, c2), lambda i: (0, 0)),
            pl.BlockSpec((1, c2), lambda i: (0, 0)),
        ],
        out_specs=pl.BlockSpec((hw, nb, c2), lambda i: (0, i, 0)),
        compiler_params=pltpu.CompilerParams(
            dimension_semantics=("parallel",)),
    )(xv, w1, w2r,
      s1.reshape(1, cp).astype(jnp.float32),
      b1.reshape(1, cp).astype(jnp.float32),
      s2.reshape(1, c2).astype(jnp.float32),
      b2.reshape(1, c2).astype(jnp.float32))
    return jnp.transpose(out.reshape(h, w, n, c2), (2, 3, 0, 1))


# speedup vs baseline: 1.1201x; 1.1201x over previous
import functools

import jax
import jax.numpy as jnp
from jax.experimental import pallas as pl
from jax.experimental.pallas import tpu as pltpu


def _win5_ax0(x):
    a = x.shape[0]
    m1 = jnp.maximum(x[0:a - 1], x[1:a])
    m2 = jnp.maximum(m1[0:a - 3], m1[2:a - 1])
    return jnp.maximum(m2[0:a - 4], x[4:a])


def _win5_ax1(x):
    b = x.shape[1]
    m1 = jnp.maximum(x[:, 0:b - 1], x[:, 1:b])
    m2 = jnp.maximum(m1[:, 0:b - 3], m1[:, 2:b - 1])
    return jnp.maximum(m2[:, 0:b - 4], x[:, 4:b])


def _pool5(x):
    return _win5_ax1(_win5_ax0(x))


def _spp_kernel(h, w, x_ref, w1_ref, w2_ref, s1_ref, b1_ref, s2_ref,
                b2_ref, o_ref):
    nb = x_ref.shape[1]
    c1 = x_ref.shape[2]
    m = h * w * nb
    xm = x_ref[...].reshape(m, c1)
    y = jnp.dot(xm, w1_ref[...], preferred_element_type=jnp.float32)
    y = y * s1_ref[...] + b1_ref[...]
    y = y * jax.nn.sigmoid(y)
    c = y.shape[-1]

    y4 = y.reshape(h, w, nb, c)
    neg_rows = jnp.full((6, w, nb, c), -jnp.inf, jnp.float32)
    yp = jnp.concatenate([neg_rows, y4, neg_rows], axis=0)
    neg_cols = jnp.full((h + 12, 6, nb, c), -jnp.inf, jnp.float32)
    yp = jnp.concatenate([neg_cols, yp, neg_cols], axis=1)
    q1 = _pool5(yp)
    q2 = _pool5(q1)
    q3 = _pool5(q2)
    p5 = q1[4:4 + h, 4:4 + w].reshape(m, c)
    p9 = q2[2:2 + h, 2:2 + w].reshape(m, c)
    p13 = q3.reshape(m, c)

    acc = jnp.dot(y, w2_ref[0], preferred_element_type=jnp.float32)
    acc = acc + jnp.dot(p5, w2_ref[1], preferred_element_type=jnp.float32)
    acc = acc + jnp.dot(p9, w2_ref[2], preferred_element_type=jnp.float32)
    acc = acc + jnp.dot(p13, w2_ref[3], preferred_element_type=jnp.float32)
    z = acc * s2_ref[...] + b2_ref[...]
    z = z * jax.nn.sigmoid(z)
    o_ref[...] = z.reshape(o_ref.shape).astype(o_ref.dtype)


@jax.jit
def kernel(x, w1, s1, b1, w2, s2, b2):
    n, c1, h, w = x.shape
    cp = w1.shape[1]
    c2 = w2.shape[1]
    hw = h * w
    nb = 8
    while n % nb:
        nb -= 1
    xv = jnp.transpose(x, (2, 3, 0, 1)).reshape(hw, n, c1)
    w2r = w2.reshape(4, cp, c2)
    out = pl.pallas_call(
        functools.partial(_spp_kernel, h, w),
        out_shape=jax.ShapeDtypeStruct((hw, n, c2), x.dtype),
        grid=(n // nb,),
        in_specs=[
            pl.BlockSpec((hw, nb, c1), lambda i: (0, i, 0)),
            pl.BlockSpec((c1, cp), lambda i: (0, 0)),
            pl.BlockSpec((4, cp, c2), lambda i: (0, 0, 0)),
            pl.BlockSpec((1, cp), lambda i: (0, 0)),
            pl.BlockSpec((1, cp), lambda i: (0, 0)),
            pl.BlockSpec((1, c2), lambda i: (0, 0)),
            pl.BlockSpec((1, c2), lambda i: (0, 0)),
        ],
        out_specs=pl.BlockSpec((hw, nb, c2), lambda i: (0, i, 0)),
        compiler_params=pltpu.CompilerParams(
            dimension_semantics=("parallel",)),
    )(xv, w1, w2r,
      s1.reshape(1, cp).astype(jnp.float32),
      b1.reshape(1, cp).astype(jnp.float32),
      s2.reshape(1, c2).astype(jnp.float32),
      b2.reshape(1, c2).astype(jnp.float32))
    return jnp.transpose(out.reshape(h, w, n, c2), (2, 3, 0, 1))
